# baseline (speedup 1.0000x reference)
import functools

import jax
import jax.numpy as jnp
from jax import lax
from jax.experimental import pallas as pl
from jax.experimental.pallas import tpu as pltpu
from jax.experimental.pallas import tpu_sc as plsc

_N = 10000
_E = 10000
_G = 256
_NP = 10240
_EP = 10240
_NW = 32
_NCH = 4
_CH = 80
_SLICE = _NP // 16
_PR = 384
_PSL = _PR // 16


def _feat_body(x_ref, m1_ref, b1_ref, w2_ref, b2_ref, wfc_ref, bfc_ref,
               degt_ref, wg1_ref, z_ref, dis_ref):
    b = x_ref.shape[0]
    x1 = jnp.broadcast_to(b1_ref[...], (b, 1984))
    for i in range(6):
        x1 = x1 + lax.dot_general(x_ref[:, i, :], m1_ref[i],
                                  (((1,), (0,)), ((), ())),
                                  preferred_element_type=jnp.float32)
    x1 = jnp.maximum(x1, 0.0)
    acc = jnp.broadcast_to(bfc_ref[...], (b, 128))
    for g in range(15):
        xg = x1[:, 128 * g:128 * g + 192]
        y2 = jnp.maximum(lax.dot_general(xg, w2_ref[...],
                                         (((1,), (0,)), ((), ())),
                                         preferred_element_type=jnp.float32)
                         + b2_ref[...], 0.0)
        acc = acc + lax.dot_general(y2, wfc_ref[g], (((1,), (0,)), ((), ())),
                                    preferred_element_type=jnp.float32)
    dis = lax.rsqrt(degt_ref[:, 0:1] + degt_ref[:, 1:2] + 1.0)
    dis_ref[...] = dis
    z_ref[...] = dis * lax.dot_general(acc, wg1_ref[...], (((1,), (1,)), ((), ())),
                                       preferred_element_type=jnp.float32)


def _gcnmid_body(acc_ref, z1_ref, dis_ref, wg2_ref, bg1_ref, z2_ref):
    s = acc_ref[0] + acc_ref[1] + z1_ref[...]
    h = jnp.maximum(dis_ref[...] * s + bg1_ref[...], 0.0)
    z2_ref[...] = dis_ref[...] * lax.dot_general(
        h, wg2_ref[...], (((1,), (1,)), ((), ())),
        preferred_element_type=jnp.float32)


def _gcnout_body(acc_ref, z2_ref, dis_ref, bg2_ref, x2_ref):
    x2_ref[...] = dis_ref[...] * (acc_ref[0] + acc_ref[1] + z2_ref[...]) \
        + bg2_ref[...]


def _head_body(q_ref, wf1_ref, bf1_ref, wf2_ref, bf2_ref, o_ref):
    pooled = q_ref[0, 0:_G, :] + q_ref[1, 0:_G, :]
    h = jnp.maximum(lax.dot_general(pooled, wf1_ref[...], (((1,), (1,)), ((), ())),
                                    preferred_element_type=jnp.float32)
                    + bf1_ref[...], 0.0)
    o_ref[...] = lax.dot_general(h, wf2_ref[...], (((1,), (1,)), ((), ())),
                                 preferred_element_type=jnp.float32) + bf2_ref[...]


_MESH = plsc.VectorSubcoreMesh(core_axis_name="c", subcore_axis_name="s")


def _zero_ref(ref, nrows, ncols):
    z16 = jnp.zeros((16,), jnp.float32)

    def rowfn(i, _):
        def colfn(j, _):
            ref[i, pl.ds(j * 16, 16)] = z16
            return 0
        return lax.fori_loop(0, ncols // 16, colfn, 0)
    lax.fori_loop(0, nrows, rowfn, 0)


@functools.partial(
    pl.kernel,
    out_type=jax.ShapeDtypeStruct((2, _NP), jnp.float32),
    mesh=_MESH,
    scratch_types=[
        pltpu.VMEM((_NCH, _CH), jnp.int32),
        pltpu.VMEM((_CH,), jnp.float32),
        pltpu.VMEM((_SLICE,), jnp.float32),
        pltpu.VMEM_SHARED((_NP,), jnp.float32),
    ],
)
def _deg_kernel(col_hbm, out_hbm, col_v, ones_v, buf_v, acc_sh):
    cid = lax.axis_index("c")
    sid = lax.axis_index("s")
    wid = sid * 2 + cid

    z16 = jnp.zeros((16,), jnp.float32)
    o16 = jnp.ones((16,), jnp.float32)

    def zf(i, _):
        buf_v[pl.ds(i * 16, 16)] = z16
        return 0
    lax.fori_loop(0, _SLICE // 16, zf, 0)
    pltpu.sync_copy(buf_v, acc_sh.at[pl.ds(sid * _SLICE, _SLICE)])

    def of(i, _):
        ones_v[pl.ds(i * 16, 16)] = o16
        return 0
    lax.fori_loop(0, _CH // 16, of, 0)

    pltpu.sync_copy(col_hbm.at[wid], col_v)
    plsc.subcore_barrier()
    for j in range(_NCH):
        pltpu.sync_copy(ones_v, acc_sh.at[col_v.at[j]], add=True)
    plsc.subcore_barrier()
    pltpu.sync_copy(acc_sh.at[pl.ds(sid * _SLICE, _SLICE)], buf_v)
    pltpu.sync_copy(buf_v, out_hbm.at[cid, pl.ds(sid * _SLICE, _SLICE)])


@functools.partial(
    pl.kernel,
    out_type=jax.ShapeDtypeStruct((2, _NP, 128), jnp.float32),
    mesh=_MESH,
    scratch_types=[
        pltpu.VMEM((_NCH, _CH), jnp.int32),
        pltpu.VMEM((_NCH, _CH), jnp.int32),
        pltpu.VMEM((_CH, 128), jnp.float32),
        pltpu.VMEM((_CH, 128), jnp.float32),
        pltpu.VMEM_SHARED((_NP, 128), jnp.float32),
        pltpu.SemaphoreType.DMA,
        pltpu.SemaphoreType.DMA,
    ],
)
def _edge_scatter_kernel(z_hbm, row_hbm, col_hbm, out_hbm,
                         row_v, col_v, buf0, buf1, acc_sh, sem0, sem1):
    cid = lax.axis_index("c")
    sid = lax.axis_index("s")
    wid = sid * 2 + cid

    _zero_ref(buf0, _CH, 128)

    def zf(k, _):
        pltpu.sync_copy(buf0, acc_sh.at[pl.ds(sid * _SLICE + k * _CH, _CH)])
        return 0
    lax.fori_loop(0, _SLICE // _CH, zf, 0)

    pltpu.sync_copy(row_hbm.at[wid], row_v)
    pltpu.sync_copy(col_hbm.at[wid], col_v)
    plsc.subcore_barrier()

    bufs = (buf0, buf1)
    sems = (sem0, sem1)
    cps = [None] * _NCH
    cps[0] = pltpu.async_copy(z_hbm.at[row_v.at[0]], buf0, sem0)
    for j in range(_NCH):
        if j + 1 < _NCH:
            cps[j + 1] = pltpu.async_copy(z_hbm.at[row_v.at[j + 1]],
                                          bufs[(j + 1) % 2], sems[(j + 1) % 2])
        cps[j].wait()
        pltpu.sync_copy(bufs[j % 2], acc_sh.at[col_v.at[j]], add=True)
    plsc.subcore_barrier()

    def wf(k, _):
        pltpu.sync_copy(acc_sh.at[pl.ds(sid * _SLICE + k * _CH, _CH)], buf0)
        pltpu.sync_copy(buf0, out_hbm.at[cid, pl.ds(sid * _SLICE + k * _CH, _CH)])
        return 0
    lax.fori_loop(0, _SLICE // _CH, wf, 0)


@functools.partial(
    pl.kernel,
    out_type=jax.ShapeDtypeStruct((2, _PR, 128), jnp.float32),
    mesh=_MESH,
    scratch_types=[
        pltpu.VMEM((_NCH, _CH), jnp.int32),
        pltpu.VMEM((_NCH, _CH), jnp.int32),
        pltpu.VMEM((_CH, 128), jnp.float32),
        pltpu.VMEM((_PSL, 128), jnp.float32),
        pltpu.VMEM_SHARED((_PR, 128), jnp.float32),
        pltpu.SemaphoreType.DMA,
    ],
)
def _pool_kernel(x2_hbm, row_hbm, batch_hbm, out_hbm,
                 row_v, g_v, buf_v, stg_v, acc_sh, sem):
    cid = lax.axis_index("c")
    sid = lax.axis_index("s")
    wid = sid * 2 + cid

    _zero_ref(stg_v, _PSL, 128)
    pltpu.sync_copy(stg_v, acc_sh.at[pl.ds(sid * _PSL, _PSL)])

    pltpu.sync_copy(row_hbm.at[wid], row_v)
    for j in range(_NCH):
        pltpu.async_copy(batch_hbm.at[row_v.at[j]], g_v.at[j], sem).wait()
    plsc.subcore_barrier()

    for j in range(_NCH):
        @pl.when(wid * _NCH * _CH + j * _CH < _N)
        def _():
            pltpu.sync_copy(x2_hbm.at[pl.ds(wid * _NCH * _CH + j * _CH, _CH)],
                            buf_v)
            pltpu.sync_copy(buf_v, acc_sh.at[g_v.at[j]], add=True)
    plsc.subcore_barrier()

    pltpu.sync_copy(acc_sh.at[pl.ds(sid * _PSL, _PSL)], stg_v)
    pltpu.sync_copy(stg_v, out_hbm.at[cid, pl.ds(sid * _PSL, _PSL)])


def kernel(edge_attr, edge_index, batch, W1, b1, W2, b2, Wfc, bfc,
           Wg1, bg1, Wg2, bg2, Wf1, bf1, Wf2, bf2):
    f32 = jnp.float32
    cols = jnp.arange(124)
    colmask = ((cols >= 2) & (cols < 122)).astype(f32)
    m1 = jnp.zeros((6, 120, 124, 16), f32)
    for k in range(5):
        ek = jnp.eye(120, 124, k=4 - k, dtype=f32) * colmask[None, :]
        m1 = m1 + jnp.einsum('hp,oi->ihpo', ek, W1[:, :, k])
    m1 = m1.reshape(6, 120, 1984)
    b1h = (jnp.tile(b1[None, :], (124, 1)) * colmask[:, None]).reshape(1, 1984)
    w2blk = jnp.zeros((12, 16, 8, 32), f32)
    for k in range(5):
        e2 = jnp.eye(8, 12, k=k, dtype=f32)
        w2blk = w2blk + jnp.einsum('hp,ci->pihc', e2, W2[:, :, k])
    w2blk = w2blk.reshape(192, 256)
    b2h = jnp.tile(b2[None, :], (8, 1)).reshape(1, 256)
    wfc_r = Wfc.reshape(128, 32, 120).transpose(2, 1, 0).reshape(15, 256, 128)
    row = edge_index[0]
    col = edge_index[1]
    row_p = jnp.concatenate([row, jnp.zeros((_EP - _E,), jnp.int32)]) \
        .reshape(_NW, _NCH, _CH)
    col_p = jnp.concatenate([col, jnp.full((_EP - _E,), _N, jnp.int32)]) \
        .reshape(_NW, _NCH, _CH)

    degs = _deg_kernel(col_p)
    degt = jnp.transpose(degs, (1, 0))

    bb = 400
    z1, dis = pl.pallas_call(
        _feat_body,
        grid=(_N // bb,),
        in_specs=[
            pl.BlockSpec((bb, 6, 120), lambda i: (i, 0, 0)),
            pl.BlockSpec((6, 120, 1984), lambda i: (0, 0, 0)),
            pl.BlockSpec((1, 1984), lambda i: (0, 0)),
            pl.BlockSpec((192, 256), lambda i: (0, 0)),
            pl.BlockSpec((1, 256), lambda i: (0, 0)),
            pl.BlockSpec((15, 256, 128), lambda i: (0, 0, 0)),
            pl.BlockSpec((1, 128), lambda i: (0, 0)),
            pl.BlockSpec((bb, 2), lambda i: (i, 0)),
            pl.BlockSpec((128, 128), lambda i: (0, 0)),
        ],
        out_specs=[pl.BlockSpec((bb, 128), lambda i: (i, 0)),
                   pl.BlockSpec((bb, 1), lambda i: (i, 0))],
        out_shape=[jax.ShapeDtypeStruct((_N, 128), f32),
                   jax.ShapeDtypeStruct((_N, 1), f32)],
    )(edge_attr, m1, b1h, w2blk, b2h, wfc_r, bfc.reshape(1, 128), degt, Wg1)

    acc1 = _edge_scatter_kernel(z1, row_p, col_p)

    rb = 1000
    z2 = pl.pallas_call(
        _gcnmid_body,
        grid=(_N // rb,),
        in_specs=[
            pl.BlockSpec((2, rb, 128), lambda i: (0, i, 0)),
            pl.BlockSpec((rb, 128), lambda i: (i, 0)),
            pl.BlockSpec((rb, 1), lambda i: (i, 0)),
            pl.BlockSpec((128, 128), lambda i: (0, 0)),
            pl.BlockSpec((1, 128), lambda i: (0, 0)),
        ],
        out_specs=pl.BlockSpec((rb, 128), lambda i: (i, 0)),
        out_shape=jax.ShapeDtypeStruct((_N, 128), f32),
    )(acc1, z1, dis, Wg2, bg1.reshape(1, 128))

    acc2 = _edge_scatter_kernel(z2, row_p, col_p)

    x2 = pl.pallas_call(
        _gcnout_body,
        grid=(_N // rb,),
        in_specs=[
            pl.BlockSpec((2, rb, 128), lambda i: (0, i, 0)),
            pl.BlockSpec((rb, 128), lambda i: (i, 0)),
            pl.BlockSpec((rb, 1), lambda i: (i, 0)),
            pl.BlockSpec((1, 128), lambda i: (0, 0)),
        ],
        out_specs=pl.BlockSpec((rb, 128), lambda i: (i, 0)),
        out_shape=jax.ShapeDtypeStruct((_N, 128), f32),
    )(acc2, z2, dis, bg2.reshape(1, 128))

    q = _pool_kernel(x2, row_p, batch)

    out = pl.pallas_call(
        _head_body,
        in_specs=[
            pl.BlockSpec((2, _PR, 128), lambda: (0, 0, 0)),
            pl.BlockSpec((256, 128), lambda: (0, 0)),
            pl.BlockSpec((1, 256), lambda: (0, 0)),
            pl.BlockSpec((128, 256), lambda: (0, 0)),
            pl.BlockSpec((1, 128), lambda: (0, 0)),
        ],
        out_specs=pl.BlockSpec((_G, 128), lambda: (0, 0)),
        out_shape=jax.ShapeDtypeStruct((_G, 128), f32),
    )(q, Wf1, bf1.reshape(1, 256), Wf2, bf2.reshape(1, 128))
    return out


# speedup vs baseline: 1.0748x; 1.0748x over previous
import functools

import jax
import jax.numpy as jnp
from jax import lax
from jax.experimental import pallas as pl
from jax.experimental.pallas import tpu as pltpu
from jax.experimental.pallas import tpu_sc as plsc

_N = 10000
_E = 10000
_G = 256
_NP = 10240
_EP = 10240
_NW = 32
_NCH = 4
_CH = 80
_SLICE = _NP // 16
_PR = 384
_PSL = _PR // 16


def _feat_body(x_ref, m1_ref, b1_ref, w2_ref, b2_ref, wfc_ref, bfc_ref,
               degt_ref, wg1_ref, z_ref, dis_ref):
    b = x_ref.shape[0]
    bf16 = jnp.bfloat16
    x1 = jnp.maximum(lax.dot_general(x_ref[...].astype(bf16), m1_ref[...],
                                     (((1,), (0,)), ((), ())),
                                     preferred_element_type=jnp.float32)
                     + b1_ref[...], 0.0)
    x1 = x1.astype(bf16)
    acc = jnp.broadcast_to(bfc_ref[...], (b, 128))
    for g in range(5):
        xg = x1[:, 384 * g:384 * g + 448]
        y2 = jnp.maximum(lax.dot_general(xg, w2_ref[...],
                                         (((1,), (0,)), ((), ())),
                                         preferred_element_type=jnp.float32)
                         + b2_ref[...], 0.0)
        acc = acc + lax.dot_general(y2.astype(bf16), wfc_ref[g],
                                    (((1,), (0,)), ((), ())),
                                    preferred_element_type=jnp.float32)
    dis = lax.rsqrt(degt_ref[:, 0:1] + degt_ref[:, 1:2] + 1.0)
    dis_ref[...] = dis
    z_ref[...] = dis * lax.dot_general(acc, wg1_ref[...], (((1,), (1,)), ((), ())),
                                       preferred_element_type=jnp.float32)


def _gcnmid_body(acc_ref, z1_ref, dis_ref, wg2_ref, bg1_ref, z2_ref):
    s = acc_ref[0] + acc_ref[1] + z1_ref[...]
    h = jnp.maximum(dis_ref[...] * s + bg1_ref[...], 0.0)
    z2_ref[...] = dis_ref[...] * lax.dot_general(
        h, wg2_ref[...], (((1,), (1,)), ((), ())),
        preferred_element_type=jnp.float32)


def _gcnout_body(acc_ref, z2_ref, dis_ref, bg2_ref, x2_ref):
    x2_ref[...] = dis_ref[...] * (acc_ref[0] + acc_ref[1] + z2_ref[...]) \
        + bg2_ref[...]


def _head_body(q_ref, wf1_ref, bf1_ref, wf2_ref, bf2_ref, o_ref):
    pooled = q_ref[0, 0:_G, :] + q_ref[1, 0:_G, :]
    h = jnp.maximum(lax.dot_general(pooled, wf1_ref[...], (((1,), (1,)), ((), ())),
                                    preferred_element_type=jnp.float32)
                    + bf1_ref[...], 0.0)
    o_ref[...] = lax.dot_general(h, wf2_ref[...], (((1,), (1,)), ((), ())),
                                 preferred_element_type=jnp.float32) + bf2_ref[...]


_MESH = plsc.VectorSubcoreMesh(core_axis_name="c", subcore_axis_name="s")


def _zero_ref(ref, nrows, ncols):
    z16 = jnp.zeros((16,), jnp.float32)

    def rowfn(i, _):
        def colfn(j, _):
            ref[i, pl.ds(j * 16, 16)] = z16
            return 0
        return lax.fori_loop(0, ncols // 16, colfn, 0)
    lax.fori_loop(0, nrows, rowfn, 0)


@functools.partial(
    pl.kernel,
    out_type=jax.ShapeDtypeStruct((2, _NP), jnp.float32),
    mesh=_MESH,
    scratch_types=[
        pltpu.VMEM((_NCH, _CH), jnp.int32),
        pltpu.VMEM((_CH,), jnp.float32),
        pltpu.VMEM((_SLICE,), jnp.float32),
        pltpu.VMEM_SHARED((_NP,), jnp.float32),
    ],
)
def _deg_kernel(col_hbm, out_hbm, col_v, ones_v, buf_v, acc_sh):
    cid = lax.axis_index("c")
    sid = lax.axis_index("s")
    wid = sid * 2 + cid

    z16 = jnp.zeros((16,), jnp.float32)
    o16 = jnp.ones((16,), jnp.float32)

    def zf(i, _):
        buf_v[pl.ds(i * 16, 16)] = z16
        return 0
    lax.fori_loop(0, _SLICE // 16, zf, 0)
    pltpu.sync_copy(buf_v, acc_sh.at[pl.ds(sid * _SLICE, _SLICE)])

    def of(i, _):
        ones_v[pl.ds(i * 16, 16)] = o16
        return 0
    lax.fori_loop(0, _CH // 16, of, 0)

    pltpu.sync_copy(col_hbm.at[wid], col_v)
    plsc.subcore_barrier()
    for j in range(_NCH):
        pltpu.sync_copy(ones_v, acc_sh.at[col_v.at[j]], add=True)
    plsc.subcore_barrier()
    pltpu.sync_copy(acc_sh.at[pl.ds(sid * _SLICE, _SLICE)], buf_v)
    pltpu.sync_copy(buf_v, out_hbm.at[cid, pl.ds(sid * _SLICE, _SLICE)])


@functools.partial(
    pl.kernel,
    out_type=jax.ShapeDtypeStruct((2, _NP, 128), jnp.float32),
    mesh=_MESH,
    scratch_types=[
        pltpu.VMEM((_NCH, _CH), jnp.int32),
        pltpu.VMEM((_NCH, _CH), jnp.int32),
        pltpu.VMEM((_CH, 128), jnp.float32),
        pltpu.VMEM((_CH, 128), jnp.float32),
        pltpu.VMEM_SHARED((_NP, 128), jnp.float32),
        pltpu.SemaphoreType.DMA,
        pltpu.SemaphoreType.DMA,
    ],
)
def _edge_scatter_kernel(z_hbm, row_hbm, col_hbm, out_hbm,
                         row_v, col_v, buf0, buf1, acc_sh, sem0, sem1):
    cid = lax.axis_index("c")
    sid = lax.axis_index("s")
    wid = sid * 2 + cid

    _zero_ref(buf0, _CH, 128)

    def zf(k, _):
        pltpu.sync_copy(buf0, acc_sh.at[pl.ds(sid * _SLICE + k * _CH, _CH)])
        return 0
    lax.fori_loop(0, _SLICE // _CH, zf, 0)

    pltpu.sync_copy(row_hbm.at[wid], row_v)
    pltpu.sync_copy(col_hbm.at[wid], col_v)
    plsc.subcore_barrier()

    bufs = (buf0, buf1)
    sems = (sem0, sem1)
    cps = [None] * _NCH
    cps[0] = pltpu.async_copy(z_hbm.at[row_v.at[0]], buf0, sem0)
    for j in range(_NCH):
        if j + 1 < _NCH:
            cps[j + 1] = pltpu.async_copy(z_hbm.at[row_v.at[j + 1]],
                                          bufs[(j + 1) % 2], sems[(j + 1) % 2])
        cps[j].wait()
        pltpu.sync_copy(bufs[j % 2], acc_sh.at[col_v.at[j]], add=True)
    plsc.subcore_barrier()

    def wf(k, _):
        pltpu.sync_copy(acc_sh.at[pl.ds(sid * _SLICE + k * _CH, _CH)], buf0)
        pltpu.sync_copy(buf0, out_hbm.at[cid, pl.ds(sid * _SLICE + k * _CH, _CH)])
        return 0
    lax.fori_loop(0, _SLICE // _CH, wf, 0)


@functools.partial(
    pl.kernel,
    out_type=jax.ShapeDtypeStruct((2, _PR, 128), jnp.float32),
    mesh=_MESH,
    scratch_types=[
        pltpu.VMEM((_NCH, _CH), jnp.int32),
        pltpu.VMEM((_NCH, _CH), jnp.int32),
        pltpu.VMEM((_CH, 128), jnp.float32),
        pltpu.VMEM((_PSL, 128), jnp.float32),
        pltpu.VMEM_SHARED((_PR, 128), jnp.float32),
        pltpu.SemaphoreType.DMA,
    ],
)
def _pool_kernel(x2_hbm, row_hbm, batch_hbm, out_hbm,
                 row_v, g_v, buf_v, stg_v, acc_sh, sem):
    cid = lax.axis_index("c")
    sid = lax.axis_index("s")
    wid = sid * 2 + cid

    _zero_ref(stg_v, _PSL, 128)
    pltpu.sync_copy(stg_v, acc_sh.at[pl.ds(sid * _PSL, _PSL)])

    pltpu.sync_copy(row_hbm.at[wid], row_v)
    for j in range(_NCH):
        pltpu.async_copy(batch_hbm.at[row_v.at[j]], g_v.at[j], sem).wait()
    plsc.subcore_barrier()

    for j in range(_NCH):
        @pl.when(wid * _NCH * _CH + j * _CH < _N)
        def _():
            pltpu.sync_copy(x2_hbm.at[pl.ds(wid * _NCH * _CH + j * _CH, _CH)],
                            buf_v)
            pltpu.sync_copy(buf_v, acc_sh.at[g_v.at[j]], add=True)
    plsc.subcore_barrier()

    pltpu.sync_copy(acc_sh.at[pl.ds(sid * _PSL, _PSL)], stg_v)
    pltpu.sync_copy(stg_v, out_hbm.at[cid, pl.ds(sid * _PSL, _PSL)])


def kernel(edge_attr, edge_index, batch, W1, b1, W2, b2, Wfc, bfc,
           Wg1, bg1, Wg2, bg2, Wf1, bf1, Wf2, bf2):
    f32 = jnp.float32
    xflat = edge_attr.reshape(_N, 720)
    cols = jnp.arange(124)
    colmask = ((cols >= 2) & (cols < 122)).astype(f32)
    m1 = jnp.zeros((6, 120, 124, 16), f32)
    for k in range(5):
        ek = jnp.eye(120, 124, k=4 - k, dtype=f32) * colmask[None, :]
        m1 = m1 + jnp.einsum('hp,oi->ihpo', ek, W1[:, :, k])
    m1 = m1.reshape(720, 1984).astype(jnp.bfloat16)
    b1h = (jnp.tile(b1[None, :], (124, 1)) * colmask[:, None]).reshape(1, 1984)
    w2blk = jnp.zeros((28, 16, 24, 32), f32)
    for k in range(5):
        e2 = jnp.eye(24, 28, k=k, dtype=f32)
        w2blk = w2blk + jnp.einsum('hp,ci->pihc', e2, W2[:, :, k])
    w2blk = w2blk.reshape(448, 768).astype(jnp.bfloat16)
    b2h = jnp.tile(b2[None, :], (24, 1)).reshape(1, 768)
    wfc_r = Wfc.reshape(128, 32, 120).transpose(2, 1, 0) \
        .reshape(5, 768, 128).astype(jnp.bfloat16)
    row = edge_index[0]
    col = edge_index[1]
    row_p = jnp.concatenate([row, jnp.zeros((_EP - _E,), jnp.int32)]) \
        .reshape(_NW, _NCH, _CH)
    col_p = jnp.concatenate([col, jnp.full((_EP - _E,), _N, jnp.int32)]) \
        .reshape(_NW, _NCH, _CH)

    degs = _deg_kernel(col_p)
    degt = jnp.transpose(degs, (1, 0))

    bb = 400
    z1, dis = pl.pallas_call(
        _feat_body,
        grid=(_N // bb,),
        in_specs=[
            pl.BlockSpec((bb, 720), lambda i: (i, 0)),
            pl.BlockSpec((720, 1984), lambda i: (0, 0)),
            pl.BlockSpec((1, 1984), lambda i: (0, 0)),
            pl.BlockSpec((448, 768), lambda i: (0, 0)),
            pl.BlockSpec((1, 768), lambda i: (0, 0)),
            pl.BlockSpec((5, 768, 128), lambda i: (0, 0, 0)),
            pl.BlockSpec((1, 128), lambda i: (0, 0)),
            pl.BlockSpec((bb, 2), lambda i: (i, 0)),
            pl.BlockSpec((128, 128), lambda i: (0, 0)),
        ],
        out_specs=[pl.BlockSpec((bb, 128), lambda i: (i, 0)),
                   pl.BlockSpec((bb, 1), lambda i: (i, 0))],
        out_shape=[jax.ShapeDtypeStruct((_N, 128), f32),
                   jax.ShapeDtypeStruct((_N, 1), f32)],
    )(xflat, m1, b1h, w2blk, b2h, wfc_r, bfc.reshape(1, 128), degt, Wg1)

    acc1 = _edge_scatter_kernel(z1, row_p, col_p)

    rb = 1000
    z2 = pl.pallas_call(
        _gcnmid_body,
        grid=(_N // rb,),
        in_specs=[
            pl.BlockSpec((2, rb, 128), lambda i: (0, i, 0)),
            pl.BlockSpec((rb, 128), lambda i: (i, 0)),
            pl.BlockSpec((rb, 1), lambda i: (i, 0)),
            pl.BlockSpec((128, 128), lambda i: (0, 0)),
            pl.BlockSpec((1, 128), lambda i: (0, 0)),
        ],
        out_specs=pl.BlockSpec((rb, 128), lambda i: (i, 0)),
        out_shape=jax.ShapeDtypeStruct((_N, 128), f32),
    )(acc1, z1, dis, Wg2, bg1.reshape(1, 128))

    acc2 = _edge_scatter_kernel(z2, row_p, col_p)

    x2 = pl.pallas_call(
        _gcnout_body,
        grid=(_N // rb,),
        in_specs=[
            pl.BlockSpec((2, rb, 128), lambda i: (0, i, 0)),
            pl.BlockSpec((rb, 128), lambda i: (i, 0)),
            pl.BlockSpec((rb, 1), lambda i: (i, 0)),
            pl.BlockSpec((1, 128), lambda i: (0, 0)),
        ],
        out_specs=pl.BlockSpec((rb, 128), lambda i: (i, 0)),
        out_shape=jax.ShapeDtypeStruct((_N, 128), f32),
    )(acc2, z2, dis, bg2.reshape(1, 128))

    q = _pool_kernel(x2, row_p, batch)

    out = pl.pallas_call(
        _head_body,
        in_specs=[
            pl.BlockSpec((2, _PR, 128), lambda: (0, 0, 0)),
            pl.BlockSpec((256, 128), lambda: (0, 0)),
            pl.BlockSpec((1, 256), lambda: (0, 0)),
            pl.BlockSpec((128, 256), lambda: (0, 0)),
            pl.BlockSpec((1, 128), lambda: (0, 0)),
        ],
        out_specs=pl.BlockSpec((_G, 128), lambda: (0, 0)),
        out_shape=jax.ShapeDtypeStruct((_G, 128), f32),
    )(q, Wf1, bf1.reshape(1, 256), Wf2, bf2.reshape(1, 128))
    return out


# speedup vs baseline: 1.1684x; 1.0871x over previous
import functools

import jax
import jax.numpy as jnp
from jax import lax
from jax.experimental import pallas as pl
from jax.experimental.pallas import tpu as pltpu
from jax.experimental.pallas import tpu_sc as plsc

_N = 10000
_E = 10000
_G = 256
_NP = 10240
_EP = 10240
_NW = 32
_NCH = 4
_CH = 80
_SLICE = _NP // 16
_PR = 384
_PSL = _PR // 16


def _feat_body(x_ref, m1_ref, b1_ref, w2_ref, b2_ref, wfc_ref, bfc_ref,
               degt_ref, wg1_ref, z_ref, dis_ref):
    b = x_ref.shape[0]
    x1 = jnp.maximum(lax.dot_general(x_ref[...], m1_ref[...],
                                     (((1,), (0,)), ((), ())),
                                     preferred_element_type=jnp.float32)
                     + b1_ref[...], 0.0)
    acc = jnp.broadcast_to(bfc_ref[...], (b, 128))
    for g in range(15):
        xg = x1[:, 128 * g:128 * g + 192]
        y2 = jnp.maximum(lax.dot_general(xg, w2_ref[...],
                                         (((1,), (0,)), ((), ())),
                                         preferred_element_type=jnp.float32)
                         + b2_ref[...], 0.0)
        acc = acc + lax.dot_general(y2, wfc_ref[g],
                                    (((1,), (0,)), ((), ())),
                                    preferred_element_type=jnp.float32)
    dis = lax.rsqrt(degt_ref[:, 0:1] + degt_ref[:, 1:2] + 1.0)
    dis_ref[...] = dis
    z_ref[...] = dis * lax.dot_general(acc, wg1_ref[...], (((1,), (1,)), ((), ())),
                                       preferred_element_type=jnp.float32)


def _gcnmid_body(acc_ref, z1_ref, dis_ref, wg2_ref, bg1_ref, z2_ref):
    s = acc_ref[0] + acc_ref[1] + z1_ref[...]
    h = jnp.maximum(dis_ref[...] * s + bg1_ref[...], 0.0)
    z2_ref[...] = dis_ref[...] * lax.dot_general(
        h, wg2_ref[...], (((1,), (1,)), ((), ())),
        preferred_element_type=jnp.float32)


def _gcnout_body(acc_ref, z2_ref, dis_ref, bg2_ref, x2_ref):
    x2_ref[...] = dis_ref[...] * (acc_ref[0] + acc_ref[1] + z2_ref[...]) \
        + bg2_ref[...]


def _head_body(q_ref, wf1_ref, bf1_ref, wf2_ref, bf2_ref, o_ref):
    pooled = q_ref[0, 0:_G, :] + q_ref[1, 0:_G, :]
    h = jnp.maximum(lax.dot_general(pooled, wf1_ref[...], (((1,), (1,)), ((), ())),
                                    preferred_element_type=jnp.float32)
                    + bf1_ref[...], 0.0)
    o_ref[...] = lax.dot_general(h, wf2_ref[...], (((1,), (1,)), ((), ())),
                                 preferred_element_type=jnp.float32) + bf2_ref[...]


_MESH = plsc.VectorSubcoreMesh(core_axis_name="c", subcore_axis_name="s")


def _zero_ref(ref, nrows, ncols):
    z16 = jnp.zeros((16,), jnp.float32)

    def rowfn(i, _):
        def colfn(j, _):
            ref[i, pl.ds(j * 16, 16)] = z16
            return 0
        return lax.fori_loop(0, ncols // 16, colfn, 0)
    lax.fori_loop(0, nrows, rowfn, 0)


@functools.partial(
    pl.kernel,
    out_type=jax.ShapeDtypeStruct((2, _NP), jnp.float32),
    mesh=_MESH,
    scratch_types=[
        pltpu.VMEM((_NCH, _CH), jnp.int32),
        pltpu.VMEM((_CH,), jnp.float32),
        pltpu.VMEM((_SLICE,), jnp.float32),
        pltpu.VMEM_SHARED((_NP,), jnp.float32),
    ],
)
def _deg_kernel(col_hbm, out_hbm, col_v, ones_v, buf_v, acc_sh):
    cid = lax.axis_index("c")
    sid = lax.axis_index("s")
    wid = sid * 2 + cid

    z16 = jnp.zeros((16,), jnp.float32)
    o16 = jnp.ones((16,), jnp.float32)

    def zf(i, _):
        buf_v[pl.ds(i * 16, 16)] = z16
        return 0
    lax.fori_loop(0, _SLICE // 16, zf, 0)
    pltpu.sync_copy(buf_v, acc_sh.at[pl.ds(sid * _SLICE, _SLICE)])

    def of(i, _):
        ones_v[pl.ds(i * 16, 16)] = o16
        return 0
    lax.fori_loop(0, _CH // 16, of, 0)

    pltpu.sync_copy(col_hbm.at[wid], col_v)
    plsc.subcore_barrier()
    for j in range(_NCH):
        pltpu.sync_copy(ones_v, acc_sh.at[col_v.at[j]], add=True)
    plsc.subcore_barrier()
    pltpu.sync_copy(acc_sh.at[pl.ds(sid * _SLICE, _SLICE)], buf_v)
    pltpu.sync_copy(buf_v, out_hbm.at[cid, pl.ds(sid * _SLICE, _SLICE)])


@functools.partial(
    pl.kernel,
    out_type=jax.ShapeDtypeStruct((2, _NP, 128), jnp.float32),
    mesh=_MESH,
    scratch_types=[
        pltpu.VMEM((_NCH, _CH), jnp.int32),
        pltpu.VMEM((_NCH, _CH), jnp.int32),
        pltpu.VMEM((_CH, 128), jnp.float32),
        pltpu.VMEM((_CH, 128), jnp.float32),
        pltpu.VMEM_SHARED((_NP, 128), jnp.float32),
        pltpu.SemaphoreType.DMA,
        pltpu.SemaphoreType.DMA,
    ],
)
def _edge_scatter_kernel(z_hbm, row_hbm, col_hbm, out_hbm,
                         row_v, col_v, buf0, buf1, acc_sh, sem0, sem1):
    cid = lax.axis_index("c")
    sid = lax.axis_index("s")
    wid = sid * 2 + cid

    _zero_ref(buf0, _CH, 128)

    def zf(k, _):
        pltpu.sync_copy(buf0, acc_sh.at[pl.ds(sid * _SLICE + k * _CH, _CH)])
        return 0
    lax.fori_loop(0, _SLICE // _CH, zf, 0)

    pltpu.sync_copy(row_hbm.at[wid], row_v)
    pltpu.sync_copy(col_hbm.at[wid], col_v)
    plsc.subcore_barrier()

    bufs = (buf0, buf1)
    sems = (sem0, sem1)
    cps = [None] * _NCH
    cps[0] = pltpu.async_copy(z_hbm.at[row_v.at[0]], buf0, sem0)
    for j in range(_NCH):
        if j + 1 < _NCH:
            cps[j + 1] = pltpu.async_copy(z_hbm.at[row_v.at[j + 1]],
                                          bufs[(j + 1) % 2], sems[(j + 1) % 2])
        cps[j].wait()
        pltpu.sync_copy(bufs[j % 2], acc_sh.at[col_v.at[j]], add=True)
    plsc.subcore_barrier()

    def wf(k, _):
        pltpu.sync_copy(acc_sh.at[pl.ds(sid * _SLICE + k * _CH, _CH)], buf0)
        pltpu.sync_copy(buf0, out_hbm.at[cid, pl.ds(sid * _SLICE + k * _CH, _CH)])
        return 0
    lax.fori_loop(0, _SLICE // _CH, wf, 0)


@functools.partial(
    pl.kernel,
    out_type=jax.ShapeDtypeStruct((2, _PR, 128), jnp.float32),
    mesh=_MESH,
    scratch_types=[
        pltpu.VMEM((_NCH, _CH), jnp.int32),
        pltpu.VMEM((_NCH, _CH), jnp.int32),
        pltpu.VMEM((_CH, 128), jnp.float32),
        pltpu.VMEM((_PSL, 128), jnp.float32),
        pltpu.VMEM_SHARED((_PR, 128), jnp.float32),
        pltpu.SemaphoreType.DMA,
    ],
)
def _pool_kernel(x2_hbm, row_hbm, batch_hbm, out_hbm,
                 row_v, g_v, buf_v, stg_v, acc_sh, sem):
    cid = lax.axis_index("c")
    sid = lax.axis_index("s")
    wid = sid * 2 + cid

    _zero_ref(stg_v, _PSL, 128)
    pltpu.sync_copy(stg_v, acc_sh.at[pl.ds(sid * _PSL, _PSL)])

    pltpu.sync_copy(row_hbm.at[wid], row_v)
    for j in range(_NCH):
        pltpu.async_copy(batch_hbm.at[row_v.at[j]], g_v.at[j], sem).wait()
    plsc.subcore_barrier()

    for j in range(_NCH):
        @pl.when(wid * _NCH * _CH + j * _CH < _N)
        def _():
            pltpu.sync_copy(x2_hbm.at[pl.ds(wid * _NCH * _CH + j * _CH, _CH)],
                            buf_v)
            pltpu.sync_copy(buf_v, acc_sh.at[g_v.at[j]], add=True)
    plsc.subcore_barrier()

    pltpu.sync_copy(acc_sh.at[pl.ds(sid * _PSL, _PSL)], stg_v)
    pltpu.sync_copy(stg_v, out_hbm.at[cid, pl.ds(sid * _PSL, _PSL)])


def kernel(edge_attr, edge_index, batch, W1, b1, W2, b2, Wfc, bfc,
           Wg1, bg1, Wg2, bg2, Wf1, bf1, Wf2, bf2):
    f32 = jnp.float32
    xflat = edge_attr.reshape(_N, 720).astype(jnp.bfloat16)
    cols = jnp.arange(124)
    colmask = ((cols >= 2) & (cols < 122)).astype(f32)
    m1 = jnp.zeros((6, 120, 124, 16), f32)
    for k in range(5):
        ek = jnp.eye(120, 124, k=4 - k, dtype=f32) * colmask[None, :]
        m1 = m1 + jnp.einsum('hp,oi->ihpo', ek, W1[:, :, k])
    m1 = m1.reshape(720, 1984).astype(jnp.bfloat16)
    b1h = (jnp.tile(b1[None, :], (124, 1)) * colmask[:, None]).reshape(1, 1984)
    w2blk = jnp.zeros((12, 16, 8, 32), f32)
    for k in range(5):
        e2 = jnp.eye(8, 12, k=k, dtype=f32)
        w2blk = w2blk + jnp.einsum('hp,ci->pihc', e2, W2[:, :, k])
    w2blk = w2blk.reshape(192, 256)
    b2h = jnp.tile(b2[None, :], (8, 1)).reshape(1, 256)
    wfc_r = Wfc.reshape(128, 32, 120).transpose(2, 1, 0).reshape(15, 256, 128)
    row = edge_index[0]
    col = edge_index[1]
    row_p = jnp.concatenate([row, jnp.zeros((_EP - _E,), jnp.int32)]) \
        .reshape(_NW, _NCH, _CH)
    col_p = jnp.concatenate([col, jnp.full((_EP - _E,), _N, jnp.int32)]) \
        .reshape(_NW, _NCH, _CH)

    degs = _deg_kernel(col_p)
    degt = jnp.transpose(degs, (1, 0))

    bb = 1000
    z1, dis = pl.pallas_call(
        _feat_body,
        grid=(_N // bb,),
        in_specs=[
            pl.BlockSpec((bb, 720), lambda i: (i, 0)),
            pl.BlockSpec((720, 1984), lambda i: (0, 0)),
            pl.BlockSpec((1, 1984), lambda i: (0, 0)),
            pl.BlockSpec((192, 256), lambda i: (0, 0)),
            pl.BlockSpec((1, 256), lambda i: (0, 0)),
            pl.BlockSpec((15, 256, 128), lambda i: (0, 0, 0)),
            pl.BlockSpec((1, 128), lambda i: (0, 0)),
            pl.BlockSpec((bb, 2), lambda i: (i, 0)),
            pl.BlockSpec((128, 128), lambda i: (0, 0)),
        ],
        out_specs=[pl.BlockSpec((bb, 128), lambda i: (i, 0)),
                   pl.BlockSpec((bb, 1), lambda i: (i, 0))],
        out_shape=[jax.ShapeDtypeStruct((_N, 128), f32),
                   jax.ShapeDtypeStruct((_N, 1), f32)],
    )(xflat, m1, b1h, w2blk, b2h, wfc_r, bfc.reshape(1, 128), degt, Wg1)

    acc1 = _edge_scatter_kernel(z1, row_p, col_p)

    rb = 1000
    z2 = pl.pallas_call(
        _gcnmid_body,
        grid=(_N // rb,),
        in_specs=[
            pl.BlockSpec((2, rb, 128), lambda i: (0, i, 0)),
            pl.BlockSpec((rb, 128), lambda i: (i, 0)),
            pl.BlockSpec((rb, 1), lambda i: (i, 0)),
            pl.BlockSpec((128, 128), lambda i: (0, 0)),
            pl.BlockSpec((1, 128), lambda i: (0, 0)),
        ],
        out_specs=pl.BlockSpec((rb, 128), lambda i: (i, 0)),
        out_shape=jax.ShapeDtypeStruct((_N, 128), f32),
    )(acc1, z1, dis, Wg2, bg1.reshape(1, 128))

    acc2 = _edge_scatter_kernel(z2, row_p, col_p)

    x2 = pl.pallas_call(
        _gcnout_body,
        grid=(_N // rb,),
        in_specs=[
            pl.BlockSpec((2, rb, 128), lambda i: (0, i, 0)),
            pl.BlockSpec((rb, 128), lambda i: (i, 0)),
            pl.BlockSpec((rb, 1), lambda i: (i, 0)),
            pl.BlockSpec((1, 128), lambda i: (0, 0)),
        ],
        out_specs=pl.BlockSpec((rb, 128), lambda i: (i, 0)),
        out_shape=jax.ShapeDtypeStruct((_N, 128), f32),
    )(acc2, z2, dis, bg2.reshape(1, 128))

    q = _pool_kernel(x2, row_p, batch)

    out = pl.pallas_call(
        _head_body,
        in_specs=[
            pl.BlockSpec((2, _PR, 128), lambda: (0, 0, 0)),
            pl.BlockSpec((256, 128), lambda: (0, 0)),
            pl.BlockSpec((1, 256), lambda: (0, 0)),
            pl.BlockSpec((128, 256), lambda: (0, 0)),
            pl.BlockSpec((1, 128), lambda: (0, 0)),
        ],
        out_specs=pl.BlockSpec((_G, 128), lambda: (0, 0)),
        out_shape=jax.ShapeDtypeStruct((_G, 128), f32),
    )(q, Wf1, bf1.reshape(1, 256), Wf2, bf2.reshape(1, 128))
    return out


# speedup vs baseline: 1.1750x; 1.0056x over previous
import functools

import jax
import jax.numpy as jnp
from jax import lax
from jax.experimental import pallas as pl
from jax.experimental.pallas import tpu as pltpu
from jax.experimental.pallas import tpu_sc as plsc

_N = 10000
_E = 10000
_G = 256
_NP = 10240
_EP = 10240
_NW = 32
_NCH = 4
_CH = 80
_SLICE = _NP // 16
_PR = 384
_PSL = _PR // 16


def _feat_body(x_ref, m1_ref, b1_ref, w2_ref, b2_ref, wfc_ref, bfc_ref,
               degt_ref, wg1_ref, z_ref, dis_ref):
    b = x_ref.shape[0]
    x1 = jnp.maximum(lax.dot_general(x_ref[...], m1_ref[...],
                                     (((1,), (0,)), ((), ())),
                                     preferred_element_type=jnp.float32)
                     + b1_ref[...], 0.0)
    acc = jnp.broadcast_to(bfc_ref[...], (b, 128))
    for g in range(15):
        xg = x1[:, 128 * g:128 * g + 192]
        y2 = jnp.maximum(lax.dot_general(xg, w2_ref[...],
                                         (((1,), (0,)), ((), ())),
                                         preferred_element_type=jnp.float32)
                         + b2_ref[...], 0.0)
        acc = acc + lax.dot_general(y2, wfc_ref[g],
                                    (((1,), (0,)), ((), ())),
                                    preferred_element_type=jnp.float32)
    dis = lax.rsqrt(degt_ref[:, 0:1] + degt_ref[:, 1:2] + 1.0)
    dis_ref[...] = dis
    z_ref[...] = dis * lax.dot_general(acc, wg1_ref[...], (((1,), (1,)), ((), ())),
                                       preferred_element_type=jnp.float32)


def _gcnmid_body(acc_ref, z1_ref, dis_ref, wg2_ref, bg1_ref, z2_ref):
    s = acc_ref[0] + acc_ref[1] + z1_ref[...]
    h = jnp.maximum(dis_ref[...] * s + bg1_ref[...], 0.0)
    z2_ref[...] = dis_ref[...] * lax.dot_general(
        h, wg2_ref[...], (((1,), (1,)), ((), ())),
        preferred_element_type=jnp.float32)


def _gcnout_body(acc_ref, z2_ref, dis_ref, bg2_ref, x2_ref):
    x2_ref[...] = dis_ref[...] * (acc_ref[0] + acc_ref[1] + z2_ref[...]) \
        + bg2_ref[...]


def _head_body(q_ref, wf1_ref, bf1_ref, wf2_ref, bf2_ref, o_ref):
    pooled = q_ref[0, 0:_G, :] + q_ref[1, 0:_G, :]
    h = jnp.maximum(lax.dot_general(pooled, wf1_ref[...], (((1,), (1,)), ((), ())),
                                    preferred_element_type=jnp.float32)
                    + bf1_ref[...], 0.0)
    o_ref[...] = lax.dot_general(h, wf2_ref[...], (((1,), (1,)), ((), ())),
                                 preferred_element_type=jnp.float32) + bf2_ref[...]


_MESH = plsc.VectorSubcoreMesh(core_axis_name="c", subcore_axis_name="s")


def _zero_ref(ref, nrows, ncols):
    z16 = jnp.zeros((16,), jnp.float32)

    def rowfn(i, _):
        def colfn(j, _):
            ref[i, pl.ds(j * 16, 16)] = z16
            return 0
        return lax.fori_loop(0, ncols // 16, colfn, 0)
    lax.fori_loop(0, nrows, rowfn, 0)


@functools.partial(
    pl.kernel,
    out_type=jax.ShapeDtypeStruct((2, _NP), jnp.float32),
    mesh=_MESH,
    scratch_types=[
        pltpu.VMEM((_NCH, _CH), jnp.int32),
        pltpu.VMEM((_CH,), jnp.float32),
        pltpu.VMEM((_SLICE,), jnp.float32),
        pltpu.VMEM_SHARED((_NP,), jnp.float32),
    ],
)
def _deg_kernel(col_hbm, out_hbm, col_v, ones_v, buf_v, acc_sh):
    cid = lax.axis_index("c")
    sid = lax.axis_index("s")
    wid = sid * 2 + cid

    z16 = jnp.zeros((16,), jnp.float32)
    o16 = jnp.ones((16,), jnp.float32)

    def zf(i, _):
        buf_v[pl.ds(i * 16, 16)] = z16
        return 0
    lax.fori_loop(0, _SLICE // 16, zf, 0)
    pltpu.sync_copy(buf_v, acc_sh.at[pl.ds(sid * _SLICE, _SLICE)])

    def of(i, _):
        ones_v[pl.ds(i * 16, 16)] = o16
        return 0
    lax.fori_loop(0, _CH // 16, of, 0)

    pltpu.sync_copy(col_hbm.at[wid], col_v)
    plsc.subcore_barrier()
    for j in range(_NCH):
        pltpu.sync_copy(ones_v, acc_sh.at[col_v.at[j]], add=True)
    plsc.subcore_barrier()
    pltpu.sync_copy(acc_sh.at[pl.ds(sid * _SLICE, _SLICE)], buf_v)
    pltpu.sync_copy(buf_v, out_hbm.at[cid, pl.ds(sid * _SLICE, _SLICE)])


@functools.partial(
    pl.kernel,
    out_type=jax.ShapeDtypeStruct((2, _NP, 128), jnp.float32),
    mesh=_MESH,
    scratch_types=[
        pltpu.VMEM((_NCH, _CH), jnp.int32),
        pltpu.VMEM((_NCH, _CH), jnp.int32),
        pltpu.VMEM((_CH, 128), jnp.float32),
        pltpu.VMEM((_CH, 128), jnp.float32),
        pltpu.VMEM_SHARED((_NP, 128), jnp.float32),
        pltpu.SemaphoreType.DMA,
        pltpu.SemaphoreType.DMA,
    ],
)
def _edge_scatter_kernel(z_hbm, row_hbm, col_hbm, out_hbm,
                         row_v, col_v, buf0, buf1, acc_sh, sem0, sem1):
    cid = lax.axis_index("c")
    sid = lax.axis_index("s")
    wid = sid * 2 + cid

    _zero_ref(buf0, _CH, 128)

    def zf(k, _):
        pltpu.sync_copy(buf0, acc_sh.at[pl.ds(sid * _SLICE + k * _CH, _CH)])
        return 0
    lax.fori_loop(0, _SLICE // _CH, zf, 0)

    pltpu.sync_copy(row_hbm.at[wid], row_v)
    pltpu.sync_copy(col_hbm.at[wid], col_v)
    plsc.subcore_barrier()

    bufs = (buf0, buf1)
    sems = (sem0, sem1)
    cps = [None] * _NCH
    cps[0] = pltpu.async_copy(z_hbm.at[row_v.at[0]], buf0, sem0)
    for j in range(_NCH):
        if j + 1 < _NCH:
            cps[j + 1] = pltpu.async_copy(z_hbm.at[row_v.at[j + 1]],
                                          bufs[(j + 1) % 2], sems[(j + 1) % 2])
        cps[j].wait()
        pltpu.sync_copy(bufs[j % 2], acc_sh.at[col_v.at[j]], add=True)
    plsc.subcore_barrier()

    def wf(k, _):
        pltpu.sync_copy(acc_sh.at[pl.ds(sid * _SLICE + k * _CH, _CH)], buf0)
        pltpu.sync_copy(buf0, out_hbm.at[cid, pl.ds(sid * _SLICE + k * _CH, _CH)])
        return 0
    lax.fori_loop(0, _SLICE // _CH, wf, 0)


@functools.partial(
    pl.kernel,
    out_type=jax.ShapeDtypeStruct((2, _PR, 128), jnp.float32),
    mesh=_MESH,
    scratch_types=[
        pltpu.VMEM((_NCH, _CH), jnp.int32),
        pltpu.VMEM((_NCH, _CH), jnp.int32),
        pltpu.VMEM((_CH, 128), jnp.float32),
        pltpu.VMEM((_PSL, 128), jnp.float32),
        pltpu.VMEM_SHARED((_PR, 128), jnp.float32),
        pltpu.SemaphoreType.DMA,
    ],
)
def _pool_kernel(x2_hbm, row_hbm, batch_hbm, out_hbm,
                 row_v, g_v, buf_v, stg_v, acc_sh, sem):
    cid = lax.axis_index("c")
    sid = lax.axis_index("s")
    wid = sid * 2 + cid

    _zero_ref(stg_v, _PSL, 128)
    pltpu.sync_copy(stg_v, acc_sh.at[pl.ds(sid * _PSL, _PSL)])

    pltpu.sync_copy(row_hbm.at[wid], row_v)
    for j in range(_NCH):
        pltpu.async_copy(batch_hbm.at[row_v.at[j]], g_v.at[j], sem).wait()
    plsc.subcore_barrier()

    for j in range(_NCH):
        @pl.when(wid * _NCH * _CH + j * _CH < _N)
        def _():
            pltpu.sync_copy(x2_hbm.at[pl.ds(wid * _NCH * _CH + j * _CH, _CH)],
                            buf_v)
            pltpu.sync_copy(buf_v, acc_sh.at[g_v.at[j]], add=True)
    plsc.subcore_barrier()

    pltpu.sync_copy(acc_sh.at[pl.ds(sid * _PSL, _PSL)], stg_v)
    pltpu.sync_copy(stg_v, out_hbm.at[cid, pl.ds(sid * _PSL, _PSL)])


def kernel(edge_attr, edge_index, batch, W1, b1, W2, b2, Wfc, bfc,
           Wg1, bg1, Wg2, bg2, Wf1, bf1, Wf2, bf2):
    f32 = jnp.float32
    xflat = edge_attr.reshape(_N, 720).astype(jnp.bfloat16)
    cols = jnp.arange(124)
    colmask = ((cols >= 2) & (cols < 122)).astype(f32)
    m1 = jnp.zeros((6, 120, 124, 16), f32)
    for k in range(5):
        ek = jnp.eye(120, 124, k=4 - k, dtype=f32) * colmask[None, :]
        m1 = m1 + jnp.einsum('hp,oi->ihpo', ek, W1[:, :, k])
    m1 = m1.reshape(720, 1984).astype(jnp.bfloat16)
    b1h = (jnp.tile(b1[None, :], (124, 1)) * colmask[:, None]).reshape(1, 1984)
    w2blk = jnp.zeros((12, 16, 8, 32), f32)
    for k in range(5):
        e2 = jnp.eye(8, 12, k=k, dtype=f32)
        w2blk = w2blk + jnp.einsum('hp,ci->pihc', e2, W2[:, :, k])
    w2blk = w2blk.reshape(192, 256)
    b2h = jnp.tile(b2[None, :], (8, 1)).reshape(1, 256)
    wfc_r = Wfc.reshape(128, 32, 120).transpose(2, 1, 0).reshape(15, 256, 128)
    row = edge_index[0]
    col = edge_index[1]
    row_p = jnp.concatenate([row, jnp.zeros((_EP - _E,), jnp.int32)]) \
        .reshape(_NW, _NCH, _CH)
    col_p = jnp.concatenate([col, jnp.full((_EP - _E,), _N, jnp.int32)]) \
        .reshape(_NW, _NCH, _CH)

    degs = _deg_kernel(col_p)
    degt = jnp.transpose(degs, (1, 0))

    bb = 2000
    z1, dis = pl.pallas_call(
        _feat_body,
        grid=(_N // bb,),
        in_specs=[
            pl.BlockSpec((bb, 720), lambda i: (i, 0)),
            pl.BlockSpec((720, 1984), lambda i: (0, 0)),
            pl.BlockSpec((1, 1984), lambda i: (0, 0)),
            pl.BlockSpec((192, 256), lambda i: (0, 0)),
            pl.BlockSpec((1, 256), lambda i: (0, 0)),
            pl.BlockSpec((15, 256, 128), lambda i: (0, 0, 0)),
            pl.BlockSpec((1, 128), lambda i: (0, 0)),
            pl.BlockSpec((bb, 2), lambda i: (i, 0)),
            pl.BlockSpec((128, 128), lambda i: (0, 0)),
        ],
        out_specs=[pl.BlockSpec((bb, 128), lambda i: (i, 0)),
                   pl.BlockSpec((bb, 1), lambda i: (i, 0))],
        out_shape=[jax.ShapeDtypeStruct((_N, 128), f32),
                   jax.ShapeDtypeStruct((_N, 1), f32)],
    )(xflat, m1, b1h, w2blk, b2h, wfc_r, bfc.reshape(1, 128), degt, Wg1)

    acc1 = _edge_scatter_kernel(z1, row_p, col_p)

    rb = 1000
    z2 = pl.pallas_call(
        _gcnmid_body,
        grid=(_N // rb,),
        in_specs=[
            pl.BlockSpec((2, rb, 128), lambda i: (0, i, 0)),
            pl.BlockSpec((rb, 128), lambda i: (i, 0)),
            pl.BlockSpec((rb, 1), lambda i: (i, 0)),
            pl.BlockSpec((128, 128), lambda i: (0, 0)),
            pl.BlockSpec((1, 128), lambda i: (0, 0)),
        ],
        out_specs=pl.BlockSpec((rb, 128), lambda i: (i, 0)),
        out_shape=jax.ShapeDtypeStruct((_N, 128), f32),
    )(acc1, z1, dis, Wg2, bg1.reshape(1, 128))

    acc2 = _edge_scatter_kernel(z2, row_p, col_p)

    x2 = pl.pallas_call(
        _gcnout_body,
        grid=(_N // rb,),
        in_specs=[
            pl.BlockSpec((2, rb, 128), lambda i: (0, i, 0)),
            pl.BlockSpec((rb, 128), lambda i: (i, 0)),
            pl.BlockSpec((rb, 1), lambda i: (i, 0)),
            pl.BlockSpec((1, 128), lambda i: (0, 0)),
        ],
        out_specs=pl.BlockSpec((rb, 128), lambda i: (i, 0)),
        out_shape=jax.ShapeDtypeStruct((_N, 128), f32),
    )(acc2, z2, dis, bg2.reshape(1, 128))

    q = _pool_kernel(x2, row_p, batch)

    out = pl.pallas_call(
        _head_body,
        in_specs=[
            pl.BlockSpec((2, _PR, 128), lambda: (0, 0, 0)),
            pl.BlockSpec((256, 128), lambda: (0, 0)),
            pl.BlockSpec((1, 256), lambda: (0, 0)),
            pl.BlockSpec((128, 256), lambda: (0, 0)),
            pl.BlockSpec((1, 128), lambda: (0, 0)),
        ],
        out_specs=pl.BlockSpec((_G, 128), lambda: (0, 0)),
        out_shape=jax.ShapeDtypeStruct((_G, 128), f32),
    )(q, Wf1, bf1.reshape(1, 256), Wf2, bf2.reshape(1, 128))
    return out
